# baseline (device time: 113406 ns/iter reference)
import numpy as np
import jax
import jax.numpy as jnp
from jax import lax
from jax.experimental import pallas as pl
from jax.experimental.pallas import tpu as pltpu

N_DEV = 8
B = 2
SQ_PER = 128
SQ = SQ_PER * N_DEV
D = 512
HQ = 4
DH = 64
HD = HQ * DH

_inv = 1.0 / (10000.0 ** (np.arange(0, DH, 2) / DH))
_pos = np.arange(SQ)[:, None] * _inv[None, :]
_cos = np.repeat(np.cos(_pos), 2, axis=-1)
_sin = np.repeat(np.sin(_pos), 2, axis=-1)
_COS = np.tile(_cos, (1, HQ)).astype(np.float32)
_SIN = np.tile(_sin, (1, HQ)).astype(np.float32)
_P = np.zeros((HD, HD), np.float32)
for _blk in range(HQ):
    for _i in range(0, DH, 2):
        _o = _blk * DH
        _P[_o + _i + 1, _o + _i] = -1.0
        _P[_o + _i, _o + _i + 1] = 1.0
_P = _P.astype(np.float32)


def kernel(x, Wq, Wk, Wv, Wo):
    def body(x_ref, wq_ref, wk_ref, wv_ref, wo_ref, cos_ref, sin_ref, p_ref,
             out_ref,
             xch, qf, kf, vf, ctx, rssend, rsrecv,
             ag_s_sems, ag_r_sems, rs_s_sems, rs_r_sems):
        my = lax.axis_index("i")
        left = (my + N_DEV - 1) % N_DEV
        right = (my + 1) % N_DEV

        barrier_sem = pltpu.get_barrier_semaphore()
        for nbr in (left, right):
            pl.semaphore_signal(
                barrier_sem, inc=1,
                device_id=(nbr,), device_id_type=pl.DeviceIdType.MESH,
            )
        pl.semaphore_wait(barrier_sem, 2)

        bf16 = jnp.bfloat16
        wq_b = wq_ref[...].astype(bf16)
        wk_b = wk_ref[...].astype(bf16)
        wv_b = wv_ref[...].astype(bf16)
        wo_b = wo_ref[...].astype(bf16)
        p_b = p_ref[...].astype(bf16)

        def qkv_chunk(c):
            rows = pl.ds(c * SQ_PER, SQ_PER)
            cosr = cos_ref[rows, :]
            sinr = sin_ref[rows, :]
            for b in range(B):
                xb = xch[c, b]
                q = jnp.dot(xb, wq_b, preferred_element_type=jnp.float32)
                k = jnp.dot(xb, wk_b, preferred_element_type=jnp.float32)
                v = jnp.dot(xb, wv_b, preferred_element_type=jnp.float32)
                qp = jnp.dot(q.astype(bf16), p_b,
                             preferred_element_type=jnp.float32)
                kp = jnp.dot(k.astype(bf16), p_b,
                             preferred_element_type=jnp.float32)
                qr = (q * cosr + qp * sinr).astype(bf16)
                kr = (k * cosr + kp * sinr).astype(bf16)
                vb = v.astype(bf16)
                for h in range(HQ):
                    qf[b, h, rows, :] = qr[:, h * DH:(h + 1) * DH]
                    kf[b, h, rows, :] = kr[:, h * DH:(h + 1) * DH]
                    vf[b, h, rows, :] = vb[:, h * DH:(h + 1) * DH]

        xch[my] = x_ref[...].astype(bf16)
        qkv_chunk(my)
        for h in range(N_DEV - 1):
            s = (my + N_DEV - h) % N_DEV
            r = (my + N_DEV - 1 - h) % N_DEV
            send = pltpu.make_async_remote_copy(
                src_ref=xch.at[s], dst_ref=xch.at[s],
                send_sem=ag_s_sems.at[h], recv_sem=ag_r_sems.at[h],
                device_id=(right,), device_id_type=pl.DeviceIdType.MESH,
            )
            send.start()
            recv = pltpu.make_async_remote_copy(
                src_ref=xch.at[r], dst_ref=xch.at[r],
                send_sem=ag_s_sems.at[h], recv_sem=ag_r_sems.at[h],
                device_id=(right,), device_id_type=pl.DeviceIdType.MESH,
            )
            recv.wait_recv()
            send.wait_send()
            qkv_chunk(r)

        for b in range(B):
            for h in range(HQ):
                qh = qf[b, h]
                kh = kf[b, h]
                s_ = lax.dot_general(
                    qh, kh, (((1,), (1,)), ((), ())),
                    preferred_element_type=jnp.float32,
                ) * 0.125
                m = jnp.max(s_, axis=1, keepdims=True)
                e = jnp.exp(s_ - m)
                den = jnp.sum(e, axis=1, keepdims=True)
                w = (e / den).astype(bf16)
                ctx[b, h] = jnp.dot(
                    w, vf[b, h], preferred_element_type=jnp.float32
                ).astype(bf16)

        def pout(c):
            rows = pl.ds(c * SQ_PER, SQ_PER)
            res = []
            for b in range(B):
                acc = None
                for h in range(HQ):
                    t = jnp.dot(ctx[b, h, rows, :],
                                wo_b[h * DH:(h + 1) * DH, :],
                                preferred_element_type=jnp.float32)
                    acc = t if acc is None else acc + t
                res.append(acc)
            return res

        p0 = pout((my + N_DEV - 1) % N_DEV)
        rssend[0, 0] = p0[0]
        rssend[0, 1] = p0[1]
        for h in range(N_DEV - 1):
            send = pltpu.make_async_remote_copy(
                src_ref=rssend.at[h], dst_ref=rsrecv.at[h],
                send_sem=rs_s_sems.at[h], recv_sem=rs_r_sems.at[h],
                device_id=(right,), device_id_type=pl.DeviceIdType.MESH,
            )
            send.start()
            if h < N_DEV - 2:
                c = (my + N_DEV - 2 - h) % N_DEV
                pc = pout(c)
            recv = pltpu.make_async_remote_copy(
                src_ref=rssend.at[h], dst_ref=rsrecv.at[h],
                send_sem=rs_s_sems.at[h], recv_sem=rs_r_sems.at[h],
                device_id=(right,), device_id_type=pl.DeviceIdType.MESH,
            )
            recv.wait_recv()
            send.wait_send()
            if h < N_DEV - 2:
                rssend[h + 1, 0] = rsrecv[h, 0] + pc[0]
                rssend[h + 1, 1] = rsrecv[h, 1] + pc[1]

        pm = pout(my)
        out_ref[0] = rsrecv[N_DEV - 2, 0] + pm[0]
        out_ref[1] = rsrecv[N_DEV - 2, 1] + pm[1]

    cos = jnp.asarray(_COS)
    sin = jnp.asarray(_SIN)
    pmat = jnp.asarray(_P)

    return pl.pallas_call(
        body,
        out_shape=jax.ShapeDtypeStruct((B, SQ_PER, D), jnp.float32),
        in_specs=[pl.BlockSpec(memory_space=pltpu.VMEM)] * 8,
        out_specs=pl.BlockSpec(memory_space=pltpu.VMEM),
        scratch_shapes=[
            pltpu.VMEM((N_DEV, B, SQ_PER, D), jnp.bfloat16),
            pltpu.VMEM((B, HQ, SQ, DH), jnp.bfloat16),
            pltpu.VMEM((B, HQ, SQ, DH), jnp.bfloat16),
            pltpu.VMEM((B, HQ, SQ, DH), jnp.bfloat16),
            pltpu.VMEM((B, HQ, SQ, DH), jnp.bfloat16),
            pltpu.VMEM((N_DEV - 1, B, SQ_PER, D), jnp.float32),
            pltpu.VMEM((N_DEV - 1, B, SQ_PER, D), jnp.float32),
            pltpu.SemaphoreType.DMA((N_DEV - 1,)),
            pltpu.SemaphoreType.DMA((N_DEV - 1,)),
            pltpu.SemaphoreType.DMA((N_DEV - 1,)),
            pltpu.SemaphoreType.DMA((N_DEV - 1,)),
        ],
        compiler_params=pltpu.CompilerParams(
            collective_id=0,
            vmem_limit_bytes=100 * 1024 * 1024,
        ),
    )(x, Wq, Wk, Wv, Wo, cos, sin, pmat)


# device time: 94128 ns/iter; 1.2048x vs baseline; 1.2048x over previous
import numpy as np
import jax
import jax.numpy as jnp
from jax import lax
from jax.experimental import pallas as pl
from jax.experimental.pallas import tpu as pltpu

N_DEV = 8
B = 2
SQ_PER = 128
SQ = SQ_PER * N_DEV
D = 512
HQ = 4
DH = 64
HD = HQ * DH

_inv = 1.0 / (10000.0 ** (np.arange(0, DH, 2) / DH))
_pos = np.arange(SQ)[:, None] * _inv[None, :]
_cos = np.repeat(np.cos(_pos), 2, axis=-1)
_sin = np.repeat(np.sin(_pos), 2, axis=-1)
_COS = np.tile(_cos, (1, HQ)).astype(np.float32)
_SIN = np.tile(_sin, (1, HQ)).astype(np.float32)
_P = np.zeros((HD, HD), np.float32)
for _blk in range(HQ):
    for _i in range(0, DH, 2):
        _o = _blk * DH
        _P[_o + _i + 1, _o + _i] = -1.0
        _P[_o + _i, _o + _i + 1] = 1.0
_P = _P.astype(np.float32)


def kernel(x, Wq, Wk, Wv, Wo):
    def body(x_ref, wq_ref, wk_ref, wv_ref, wo_ref, cos_ref, sin_ref, p_ref,
             out_ref,
             xch, qf, kf, vf, ctx, rssend, rsrecv,
             ag_s_sems, ag_r_sems, rs_s_sems, rs_r_sems):
        my = lax.axis_index("i")
        left = (my + N_DEV - 1) % N_DEV
        right = (my + 1) % N_DEV

        barrier_sem = pltpu.get_barrier_semaphore()
        for nbr in (left, right):
            pl.semaphore_signal(
                barrier_sem, inc=1,
                device_id=(nbr,), device_id_type=pl.DeviceIdType.MESH,
            )
        pl.semaphore_wait(barrier_sem, 2)

        bf16 = jnp.bfloat16
        wq_b = wq_ref[...].astype(bf16)
        wk_b = wk_ref[...].astype(bf16)
        wv_b = wv_ref[...].astype(bf16)
        wo_b = wo_ref[...].astype(bf16)
        p_b = p_ref[...].astype(bf16)

        def qkv_chunk(c):
            rows = pl.ds(c * SQ_PER, SQ_PER)
            cosr = cos_ref[rows, :]
            sinr = sin_ref[rows, :]
            for b in range(B):
                xb = xch[c, b]
                q = jnp.dot(xb, wq_b, preferred_element_type=jnp.float32)
                k = jnp.dot(xb, wk_b, preferred_element_type=jnp.float32)
                v = jnp.dot(xb, wv_b, preferred_element_type=jnp.float32)
                qp = jnp.dot(q.astype(bf16), p_b,
                             preferred_element_type=jnp.float32)
                kp = jnp.dot(k.astype(bf16), p_b,
                             preferred_element_type=jnp.float32)
                qr = (q * cosr + qp * sinr).astype(bf16)
                kr = (k * cosr + kp * sinr).astype(bf16)
                vb = v.astype(bf16)
                for h in range(HQ):
                    qf[b, h, rows, :] = qr[:, h * DH:(h + 1) * DH]
                    kf[b, h, rows, :] = kr[:, h * DH:(h + 1) * DH]
                    vf[b, h, rows, :] = vb[:, h * DH:(h + 1) * DH]

        xch[my] = x_ref[...].astype(bf16)
        qkv_chunk(my)
        for h in range(N_DEV - 1):
            s = (my + N_DEV - h) % N_DEV
            r = (my + N_DEV - 1 - h) % N_DEV
            send = pltpu.make_async_remote_copy(
                src_ref=xch.at[s], dst_ref=xch.at[s],
                send_sem=ag_s_sems.at[h], recv_sem=ag_r_sems.at[h],
                device_id=(right,), device_id_type=pl.DeviceIdType.MESH,
            )
            send.start()
            recv = pltpu.make_async_remote_copy(
                src_ref=xch.at[r], dst_ref=xch.at[r],
                send_sem=ag_s_sems.at[h], recv_sem=ag_r_sems.at[h],
                device_id=(right,), device_id_type=pl.DeviceIdType.MESH,
            )
            recv.wait_recv()
            send.wait_send()
            qkv_chunk(r)

        for b in range(B):
            for h in range(HQ):
                qh = qf[b, h]
                kh = kf[b, h]
                s_ = lax.dot_general(
                    qh, kh, (((1,), (1,)), ((), ())),
                    preferred_element_type=jnp.float32,
                ) * 0.125
                m = jnp.max(s_, axis=1, keepdims=True)
                e = jnp.exp(s_ - m)
                den = jnp.sum(e, axis=1, keepdims=True)
                w = (e / den).astype(bf16)
                ctx[b, h] = jnp.dot(
                    w, vf[b, h], preferred_element_type=jnp.float32
                ).astype(bf16)

        def pout(c):
            rows = pl.ds(c * SQ_PER, SQ_PER)
            res = []
            for b in range(B):
                acc = None
                for h in range(HQ):
                    t = jnp.dot(ctx[b, h, rows, :],
                                wo_b[h * DH:(h + 1) * DH, :],
                                preferred_element_type=jnp.float32)
                    acc = t if acc is None else acc + t
                res.append(acc)
            return res

        p0 = pout((my + N_DEV - 1) % N_DEV)
        rssend[0, 0] = p0[0].astype(bf16)
        rssend[0, 1] = p0[1].astype(bf16)
        for h in range(N_DEV - 1):
            send = pltpu.make_async_remote_copy(
                src_ref=rssend.at[h], dst_ref=rsrecv.at[h],
                send_sem=rs_s_sems.at[h], recv_sem=rs_r_sems.at[h],
                device_id=(right,), device_id_type=pl.DeviceIdType.MESH,
            )
            send.start()
            if h < N_DEV - 2:
                c = (my + N_DEV - 2 - h) % N_DEV
                pc = pout(c)
            recv = pltpu.make_async_remote_copy(
                src_ref=rssend.at[h], dst_ref=rsrecv.at[h],
                send_sem=rs_s_sems.at[h], recv_sem=rs_r_sems.at[h],
                device_id=(right,), device_id_type=pl.DeviceIdType.MESH,
            )
            recv.wait_recv()
            send.wait_send()
            if h < N_DEV - 2:
                rssend[h + 1, 0] = (
                    rsrecv[h, 0].astype(jnp.float32) + pc[0]).astype(bf16)
                rssend[h + 1, 1] = (
                    rsrecv[h, 1].astype(jnp.float32) + pc[1]).astype(bf16)

        pm = pout(my)
        out_ref[0] = rsrecv[N_DEV - 2, 0].astype(jnp.float32) + pm[0]
        out_ref[1] = rsrecv[N_DEV - 2, 1].astype(jnp.float32) + pm[1]

    cos = jnp.asarray(_COS)
    sin = jnp.asarray(_SIN)
    pmat = jnp.asarray(_P)

    return pl.pallas_call(
        body,
        out_shape=jax.ShapeDtypeStruct((B, SQ_PER, D), jnp.float32),
        in_specs=[pl.BlockSpec(memory_space=pltpu.VMEM)] * 8,
        out_specs=pl.BlockSpec(memory_space=pltpu.VMEM),
        scratch_shapes=[
            pltpu.VMEM((N_DEV, B, SQ_PER, D), jnp.bfloat16),
            pltpu.VMEM((B, HQ, SQ, DH), jnp.bfloat16),
            pltpu.VMEM((B, HQ, SQ, DH), jnp.bfloat16),
            pltpu.VMEM((B, HQ, SQ, DH), jnp.bfloat16),
            pltpu.VMEM((B, HQ, SQ, DH), jnp.bfloat16),
            pltpu.VMEM((N_DEV - 1, B, SQ_PER, D), jnp.bfloat16),
            pltpu.VMEM((N_DEV - 1, B, SQ_PER, D), jnp.bfloat16),
            pltpu.SemaphoreType.DMA((N_DEV - 1,)),
            pltpu.SemaphoreType.DMA((N_DEV - 1,)),
            pltpu.SemaphoreType.DMA((N_DEV - 1,)),
            pltpu.SemaphoreType.DMA((N_DEV - 1,)),
        ],
        compiler_params=pltpu.CompilerParams(
            collective_id=0,
            vmem_limit_bytes=100 * 1024 * 1024,
        ),
    )(x, Wq, Wk, Wv, Wo, cos, sin, pmat)


# device time: 68427 ns/iter; 1.6573x vs baseline; 1.3756x over previous
import numpy as np
import jax
import jax.numpy as jnp
from jax import lax
from jax.experimental import pallas as pl
from jax.experimental.pallas import tpu as pltpu

N_DEV = 8
B = 2
SQ_PER = 128
SQ = SQ_PER * N_DEV
D = 512
HQ = 4
DH = 64
HD = HQ * DH
R_HOPS = 4
L_HOPS = 3

_inv = 1.0 / (10000.0 ** (np.arange(0, DH, 2) / DH))
_pos = np.arange(SQ)[:, None] * _inv[None, :]
_cos = np.repeat(np.cos(_pos), 2, axis=-1)
_sin = np.repeat(np.sin(_pos), 2, axis=-1)
_COS = np.tile(_cos, (1, HQ)).astype(np.float32)
_SIN = np.tile(_sin, (1, HQ)).astype(np.float32)
_P = np.zeros((HD, HD), np.float32)
for _blk in range(HQ):
    for _i in range(0, DH, 2):
        _o = _blk * DH
        _P[_o + _i + 1, _o + _i] = -1.0
        _P[_o + _i, _o + _i + 1] = 1.0
_P = _P.astype(np.float32)


def kernel(x, Wq, Wk, Wv, Wo):
    def body(x_ref, wq_ref, wk_ref, wv_ref, wo_ref, cos_ref, sin_ref, p_ref,
             out_ref,
             xch, qf, kf, vf, ctx, rssendR, rsrecvR, rssendL, rsrecvL,
             agRs, agRr, agLs, agLr, rsRs, rsRr, rsLs, rsLr):
        my = lax.axis_index("i")
        left = (my + N_DEV - 1) % N_DEV
        right = (my + 1) % N_DEV
        pending = []

        barrier_sem = pltpu.get_barrier_semaphore()
        for nbr in (left, right):
            pl.semaphore_signal(
                barrier_sem, inc=1,
                device_id=(nbr,), device_id_type=pl.DeviceIdType.MESH,
            )
        pl.semaphore_wait(barrier_sem, 2)

        bf16 = jnp.bfloat16
        f32 = jnp.float32
        wq_b = wq_ref[...].astype(bf16)
        wk_b = wk_ref[...].astype(bf16)
        wv_b = wv_ref[...].astype(bf16)
        wo_b = wo_ref[...].astype(bf16)
        p_b = p_ref[...].astype(bf16)

        def send_chunk(slot_c, sem_arr_s, sem_arr_r, step, dst):
            d = pltpu.make_async_remote_copy(
                src_ref=xch.at[slot_c], dst_ref=xch.at[slot_c],
                send_sem=sem_arr_s.at[step], recv_sem=sem_arr_r.at[step],
                device_id=(dst,), device_id_type=pl.DeviceIdType.MESH,
            )
            d.start()
            pending.append(d)

        def wait_chunk(slot_c, sem_arr_s, sem_arr_r, step, dst):
            d = pltpu.make_async_remote_copy(
                src_ref=xch.at[slot_c], dst_ref=xch.at[slot_c],
                send_sem=sem_arr_s.at[step], recv_sem=sem_arr_r.at[step],
                device_id=(dst,), device_id_type=pl.DeviceIdType.MESH,
            )
            d.wait_recv()

        def qkv_chunk(c):
            rows = pl.ds(c * SQ_PER, SQ_PER)
            cosr = cos_ref[rows, :]
            sinr = sin_ref[rows, :]
            for b in range(B):
                xb = xch[c, b]
                q = jnp.dot(xb, wq_b, preferred_element_type=f32)
                k = jnp.dot(xb, wk_b, preferred_element_type=f32)
                v = jnp.dot(xb, wv_b, preferred_element_type=f32)
                qp = jnp.dot(q.astype(bf16), p_b, preferred_element_type=f32)
                kp = jnp.dot(k.astype(bf16), p_b, preferred_element_type=f32)
                qr = (q * cosr + qp * sinr).astype(bf16)
                kr = (k * cosr + kp * sinr).astype(bf16)
                vb = v.astype(bf16)
                for h in range(HQ):
                    qf[b, h, rows, :] = qr[:, h * DH:(h + 1) * DH]
                    kf[b, h, rows, :] = kr[:, h * DH:(h + 1) * DH]
                    vf[b, h, rows, :] = vb[:, h * DH:(h + 1) * DH]

        xch[my] = x_ref[...].astype(bf16)
        send_chunk(my, agRs, agRr, 0, right)
        send_chunk(my, agLs, agLr, 0, left)
        qkv_chunk(my)
        for s in range(R_HOPS):
            rm = (my + N_DEV - 1 - s) % N_DEV
            wait_chunk(rm, agRs, agRr, s, right)
            if s + 1 < R_HOPS:
                send_chunk(rm, agRs, agRr, s + 1, right)
            lm = (my + 1 + s) % N_DEV
            if s < L_HOPS:
                wait_chunk(lm, agLs, agLr, s, left)
                if s + 1 < L_HOPS:
                    send_chunk(lm, agLs, agLr, s + 1, left)
            qkv_chunk(rm)
            if s < L_HOPS:
                qkv_chunk(lm)

        def att_chunk(c):
            rows = pl.ds(c * SQ_PER, SQ_PER)
            for b in range(B):
                for h in range(HQ):
                    qb = qf[b, h, rows, :]
                    s_ = lax.dot_general(
                        qb, kf[b, h], (((1,), (1,)), ((), ())),
                        preferred_element_type=f32,
                    ) * 0.125
                    m = jnp.max(s_, axis=1, keepdims=True)
                    e = jnp.exp(s_ - m)
                    den = jnp.sum(e, axis=1, keepdims=True)
                    w = (e / den).astype(bf16)
                    ctx[b, h, rows, :] = jnp.dot(
                        w, vf[b, h], preferred_element_type=f32
                    ).astype(bf16)

        def pout(c):
            rows = pl.ds(c * SQ_PER, SQ_PER)
            res = []
            for b in range(B):
                acc = None
                for h in range(HQ):
                    t = jnp.dot(ctx[b, h, rows, :],
                                wo_b[h * DH:(h + 1) * DH, :],
                                preferred_element_type=f32)
                    acc = t if acc is None else acc + t
                res.append(acc)
            return res

        def rs_send(buf_s, buf_r, sem_s, sem_r, step, dst):
            d = pltpu.make_async_remote_copy(
                src_ref=buf_s.at[step], dst_ref=buf_r.at[step],
                send_sem=sem_s.at[step], recv_sem=sem_r.at[step],
                device_id=(dst,), device_id_type=pl.DeviceIdType.MESH,
            )
            d.start()
            pending.append(d)

        def rs_wait(buf_s, buf_r, sem_s, sem_r, step, dst):
            d = pltpu.make_async_remote_copy(
                src_ref=buf_s.at[step], dst_ref=buf_r.at[step],
                send_sem=sem_s.at[step], recv_sem=sem_r.at[step],
                device_id=(dst,), device_id_type=pl.DeviceIdType.MESH,
            )
            d.wait_recv()

        att_chunk((my + R_HOPS) % N_DEV)
        att_chunk((my + N_DEV - L_HOPS) % N_DEV)
        pR = pout((my + R_HOPS) % N_DEV)
        rssendR[0, 0] = pR[0].astype(bf16)
        rssendR[0, 1] = pR[1].astype(bf16)
        rs_send(rssendR, rsrecvR, rsRs, rsRr, 0, right)
        pL = pout((my + N_DEV - L_HOPS) % N_DEV)
        rssendL[0, 0] = pL[0].astype(bf16)
        rssendL[0, 1] = pL[1].astype(bf16)
        rs_send(rssendL, rsrecvL, rsLs, rsLr, 0, left)

        for s in range(R_HOPS):
            if s < R_HOPS - 1:
                att_chunk((my + R_HOPS - 1 - s) % N_DEV)
                att_chunk((my + N_DEV - L_HOPS + 1 + s) % N_DEV)
            cr = (my + R_HOPS - 1 - s) % N_DEV
            rs_wait(rssendR, rsrecvR, rsRs, rsRr, s, right)
            if s + 1 < R_HOPS:
                pc = pout(cr)
                rssendR[s + 1, 0] = (
                    rsrecvR[s, 0].astype(f32) + pc[0]).astype(bf16)
                rssendR[s + 1, 1] = (
                    rsrecvR[s, 1].astype(f32) + pc[1]).astype(bf16)
                rs_send(rssendR, rsrecvR, rsRs, rsRr, s + 1, right)
            if s < L_HOPS:
                cl = (my + N_DEV - L_HOPS + 1 + s) % N_DEV
                rs_wait(rssendL, rsrecvL, rsLs, rsLr, s, left)
                if s + 1 < L_HOPS:
                    pc = pout(cl)
                    rssendL[s + 1, 0] = (
                        rsrecvL[s, 0].astype(f32) + pc[0]).astype(bf16)
                    rssendL[s + 1, 1] = (
                        rsrecvL[s, 1].astype(f32) + pc[1]).astype(bf16)
                    rs_send(rssendL, rsrecvL, rsLs, rsLr, s + 1, left)

        pm = pout(my)
        out_ref[0] = (pm[0] + rsrecvR[R_HOPS - 1, 0].astype(f32)
                      + rsrecvL[L_HOPS - 1, 0].astype(f32))
        out_ref[1] = (pm[1] + rsrecvR[R_HOPS - 1, 1].astype(f32)
                      + rsrecvL[L_HOPS - 1, 1].astype(f32))

        for d in pending:
            d.wait_send()

    cos = jnp.asarray(_COS)
    sin = jnp.asarray(_SIN)
    pmat = jnp.asarray(_P)

    return pl.pallas_call(
        body,
        out_shape=jax.ShapeDtypeStruct((B, SQ_PER, D), jnp.float32),
        in_specs=[pl.BlockSpec(memory_space=pltpu.VMEM)] * 8,
        out_specs=pl.BlockSpec(memory_space=pltpu.VMEM),
        scratch_shapes=[
            pltpu.VMEM((N_DEV, B, SQ_PER, D), jnp.bfloat16),
            pltpu.VMEM((B, HQ, SQ, DH), jnp.bfloat16),
            pltpu.VMEM((B, HQ, SQ, DH), jnp.bfloat16),
            pltpu.VMEM((B, HQ, SQ, DH), jnp.bfloat16),
            pltpu.VMEM((B, HQ, SQ, DH), jnp.bfloat16),
            pltpu.VMEM((R_HOPS, B, SQ_PER, D), jnp.bfloat16),
            pltpu.VMEM((R_HOPS, B, SQ_PER, D), jnp.bfloat16),
            pltpu.VMEM((L_HOPS, B, SQ_PER, D), jnp.bfloat16),
            pltpu.VMEM((L_HOPS, B, SQ_PER, D), jnp.bfloat16),
            pltpu.SemaphoreType.DMA((R_HOPS,)),
            pltpu.SemaphoreType.DMA((R_HOPS,)),
            pltpu.SemaphoreType.DMA((L_HOPS,)),
            pltpu.SemaphoreType.DMA((L_HOPS,)),
            pltpu.SemaphoreType.DMA((R_HOPS,)),
            pltpu.SemaphoreType.DMA((R_HOPS,)),
            pltpu.SemaphoreType.DMA((L_HOPS,)),
            pltpu.SemaphoreType.DMA((L_HOPS,)),
        ],
        compiler_params=pltpu.CompilerParams(
            collective_id=0,
            vmem_limit_bytes=100 * 1024 * 1024,
        ),
    )(x, Wq, Wk, Wv, Wo, cos, sin, pmat)


# device time: 53919 ns/iter; 2.1033x vs baseline; 1.2691x over previous
import numpy as np
import jax
import jax.numpy as jnp
from jax import lax
from jax.experimental import pallas as pl
from jax.experimental.pallas import tpu as pltpu

N_DEV = 8
B = 2
SQ_PER = 128
SQ = SQ_PER * N_DEV
D = 512
HQ = 4
DH = 64
HD = HQ * DH
R_HOPS = 4
L_HOPS = 3

_inv = 1.0 / (10000.0 ** (np.arange(0, DH, 2) / DH))
_pos = np.arange(SQ)[:, None] * _inv[None, :]
_cos = np.repeat(np.cos(_pos), 2, axis=-1)
_sin = np.repeat(np.sin(_pos), 2, axis=-1)
_COS = np.tile(_cos, (1, HQ)).astype(np.float32)
_SIN = np.tile(_sin, (1, HQ)).astype(np.float32)
_P = np.zeros((HD, HD), np.float32)
for _blk in range(HQ):
    for _i in range(0, DH, 2):
        _o = _blk * DH
        _P[_o + _i + 1, _o + _i] = -1.0
        _P[_o + _i, _o + _i + 1] = 1.0
_P = _P.astype(np.float32)


def kernel(x, Wq, Wk, Wv, Wo):
    def body(x_ref, wq_ref, wk_ref, wv_ref, wo_ref, cos_ref, sin_ref, p_ref,
             out_ref,
             xch, qf, kf, vf, ctx, rssendR, rsrecvR, rssendL, rsrecvL,
             agRs, agRr, agLs, agLr, rsRs, rsRr, rsLs, rsLr):
        my = lax.axis_index("i")
        left = (my + N_DEV - 1) % N_DEV
        right = (my + 1) % N_DEV
        pending = []

        barrier_sem = pltpu.get_barrier_semaphore()
        for nbr in (left, right):
            pl.semaphore_signal(
                barrier_sem, inc=1,
                device_id=(nbr,), device_id_type=pl.DeviceIdType.MESH,
            )
        pl.semaphore_wait(barrier_sem, 2)

        bf16 = jnp.bfloat16
        f32 = jnp.float32
        wq_b = (wq_ref[...] * 0.125).astype(bf16)
        wk_b = wk_ref[...].astype(bf16)
        wv_b = wv_ref[...].astype(bf16)
        wo_b = wo_ref[...].astype(bf16)
        p_b = p_ref[...].astype(bf16)

        def send_chunk(slot_c, sem_arr_s, sem_arr_r, step, dst):
            d = pltpu.make_async_remote_copy(
                src_ref=xch.at[slot_c], dst_ref=xch.at[slot_c],
                send_sem=sem_arr_s.at[step], recv_sem=sem_arr_r.at[step],
                device_id=(dst,), device_id_type=pl.DeviceIdType.MESH,
            )
            d.start()
            pending.append(d)

        def wait_chunk(slot_c, sem_arr_s, sem_arr_r, step, dst):
            d = pltpu.make_async_remote_copy(
                src_ref=xch.at[slot_c], dst_ref=xch.at[slot_c],
                send_sem=sem_arr_s.at[step], recv_sem=sem_arr_r.at[step],
                device_id=(dst,), device_id_type=pl.DeviceIdType.MESH,
            )
            d.wait_recv()

        def qkv_chunk(c):
            rows = pl.ds(c * SQ_PER, SQ_PER)
            cosr = cos_ref[rows, :]
            sinr = sin_ref[rows, :]
            for b in range(B):
                xb = xch[c, b]
                q = jnp.dot(xb, wq_b, preferred_element_type=f32)
                k = jnp.dot(xb, wk_b, preferred_element_type=f32)
                v = jnp.dot(xb, wv_b, preferred_element_type=f32)
                qp = jnp.dot(q.astype(bf16), p_b, preferred_element_type=f32)
                kp = jnp.dot(k.astype(bf16), p_b, preferred_element_type=f32)
                qr = (q * cosr + qp * sinr).astype(bf16)
                kr = (k * cosr + kp * sinr).astype(bf16)
                vb = v.astype(bf16)
                ones_col = (lax.broadcasted_iota(jnp.int32, (SQ_PER, DH), 1)
                            == 0).astype(bf16)
                for h in range(HQ):
                    qf[b, h, rows, :] = qr[:, h * DH:(h + 1) * DH]
                    kf[b, h, rows, :] = kr[:, h * DH:(h + 1) * DH]
                    vf[b, h, rows, 0:DH] = vb[:, h * DH:(h + 1) * DH]
                    vf[b, h, rows, DH:] = ones_col

        xch[my] = x_ref[...].astype(bf16)
        send_chunk(my, agRs, agRr, 0, right)
        send_chunk(my, agLs, agLr, 0, left)
        qkv_chunk(my)
        for s in range(R_HOPS):
            rm = (my + N_DEV - 1 - s) % N_DEV
            wait_chunk(rm, agRs, agRr, s, right)
            if s + 1 < R_HOPS:
                send_chunk(rm, agRs, agRr, s + 1, right)
            lm = (my + 1 + s) % N_DEV
            if s < L_HOPS:
                wait_chunk(lm, agLs, agLr, s, left)
                if s + 1 < L_HOPS:
                    send_chunk(lm, agLs, agLr, s + 1, left)
            qkv_chunk(rm)
            if s < L_HOPS:
                qkv_chunk(lm)

        def att_chunk(c):
            rows = pl.ds(c * SQ_PER, SQ_PER)
            for b in range(B):
                for h in range(HQ):
                    qb = qf[b, h, rows, :]
                    s_ = lax.dot_general(
                        qb, kf[b, h], (((1,), (1,)), ((), ())),
                        preferred_element_type=f32,
                    )
                    e = jnp.exp(s_).astype(bf16)
                    aug = jnp.dot(e, vf[b, h], preferred_element_type=f32)
                    ctx[b, h, rows, :] = (
                        aug[:, :DH] * (1.0 / aug[:, DH:DH + 1])
                    ).astype(bf16)

        def pout(c):
            rows = pl.ds(c * SQ_PER, SQ_PER)
            res = []
            for b in range(B):
                acc = None
                for h in range(HQ):
                    t = jnp.dot(ctx[b, h, rows, :],
                                wo_b[h * DH:(h + 1) * DH, :],
                                preferred_element_type=f32)
                    acc = t if acc is None else acc + t
                res.append(acc)
            return res

        def rs_send(buf_s, buf_r, sem_s, sem_r, step, dst):
            d = pltpu.make_async_remote_copy(
                src_ref=buf_s.at[step], dst_ref=buf_r.at[step],
                send_sem=sem_s.at[step], recv_sem=sem_r.at[step],
                device_id=(dst,), device_id_type=pl.DeviceIdType.MESH,
            )
            d.start()
            pending.append(d)

        def rs_wait(buf_s, buf_r, sem_s, sem_r, step, dst):
            d = pltpu.make_async_remote_copy(
                src_ref=buf_s.at[step], dst_ref=buf_r.at[step],
                send_sem=sem_s.at[step], recv_sem=sem_r.at[step],
                device_id=(dst,), device_id_type=pl.DeviceIdType.MESH,
            )
            d.wait_recv()

        att_chunk((my + R_HOPS) % N_DEV)
        att_chunk((my + N_DEV - L_HOPS) % N_DEV)
        pR = pout((my + R_HOPS) % N_DEV)
        rssendR[0, 0] = pR[0].astype(bf16)
        rssendR[0, 1] = pR[1].astype(bf16)
        rs_send(rssendR, rsrecvR, rsRs, rsRr, 0, right)
        pL = pout((my + N_DEV - L_HOPS) % N_DEV)
        rssendL[0, 0] = pL[0].astype(bf16)
        rssendL[0, 1] = pL[1].astype(bf16)
        rs_send(rssendL, rsrecvL, rsLs, rsLr, 0, left)

        for s in range(R_HOPS):
            if s < R_HOPS - 1:
                att_chunk((my + R_HOPS - 1 - s) % N_DEV)
                att_chunk((my + N_DEV - L_HOPS + 1 + s) % N_DEV)
            cr = (my + R_HOPS - 1 - s) % N_DEV
            rs_wait(rssendR, rsrecvR, rsRs, rsRr, s, right)
            if s + 1 < R_HOPS:
                pc = pout(cr)
                rssendR[s + 1, 0] = (
                    rsrecvR[s, 0].astype(f32) + pc[0]).astype(bf16)
                rssendR[s + 1, 1] = (
                    rsrecvR[s, 1].astype(f32) + pc[1]).astype(bf16)
                rs_send(rssendR, rsrecvR, rsRs, rsRr, s + 1, right)
            if s < L_HOPS:
                cl = (my + N_DEV - L_HOPS + 1 + s) % N_DEV
                rs_wait(rssendL, rsrecvL, rsLs, rsLr, s, left)
                if s + 1 < L_HOPS:
                    pc = pout(cl)
                    rssendL[s + 1, 0] = (
                        rsrecvL[s, 0].astype(f32) + pc[0]).astype(bf16)
                    rssendL[s + 1, 1] = (
                        rsrecvL[s, 1].astype(f32) + pc[1]).astype(bf16)
                    rs_send(rssendL, rsrecvL, rsLs, rsLr, s + 1, left)

        pm = pout(my)
        out_ref[0] = (pm[0] + rsrecvR[R_HOPS - 1, 0].astype(f32)
                      + rsrecvL[L_HOPS - 1, 0].astype(f32))
        out_ref[1] = (pm[1] + rsrecvR[R_HOPS - 1, 1].astype(f32)
                      + rsrecvL[L_HOPS - 1, 1].astype(f32))

        for d in pending:
            d.wait_send()

    cos = jnp.asarray(_COS)
    sin = jnp.asarray(_SIN)
    pmat = jnp.asarray(_P)

    return pl.pallas_call(
        body,
        out_shape=jax.ShapeDtypeStruct((B, SQ_PER, D), jnp.float32),
        in_specs=[pl.BlockSpec(memory_space=pltpu.VMEM)] * 8,
        out_specs=pl.BlockSpec(memory_space=pltpu.VMEM),
        scratch_shapes=[
            pltpu.VMEM((N_DEV, B, SQ_PER, D), jnp.bfloat16),
            pltpu.VMEM((B, HQ, SQ, DH), jnp.bfloat16),
            pltpu.VMEM((B, HQ, SQ, DH), jnp.bfloat16),
            pltpu.VMEM((B, HQ, SQ, 2 * DH), jnp.bfloat16),
            pltpu.VMEM((B, HQ, SQ, DH), jnp.bfloat16),
            pltpu.VMEM((R_HOPS, B, SQ_PER, D), jnp.bfloat16),
            pltpu.VMEM((R_HOPS, B, SQ_PER, D), jnp.bfloat16),
            pltpu.VMEM((L_HOPS, B, SQ_PER, D), jnp.bfloat16),
            pltpu.VMEM((L_HOPS, B, SQ_PER, D), jnp.bfloat16),
            pltpu.SemaphoreType.DMA((R_HOPS,)),
            pltpu.SemaphoreType.DMA((R_HOPS,)),
            pltpu.SemaphoreType.DMA((L_HOPS,)),
            pltpu.SemaphoreType.DMA((L_HOPS,)),
            pltpu.SemaphoreType.DMA((R_HOPS,)),
            pltpu.SemaphoreType.DMA((R_HOPS,)),
            pltpu.SemaphoreType.DMA((L_HOPS,)),
            pltpu.SemaphoreType.DMA((L_HOPS,)),
        ],
        compiler_params=pltpu.CompilerParams(
            collective_id=0,
            vmem_limit_bytes=100 * 1024 * 1024,
        ),
    )(x, Wq, Wk, Wv, Wo, cos, sin, pmat)


# device time: 50262 ns/iter; 2.2563x vs baseline; 1.0728x over previous
import numpy as np
import jax
import jax.numpy as jnp
from jax import lax
from jax.experimental import pallas as pl
from jax.experimental.pallas import tpu as pltpu

N_DEV = 8
B = 2
SQ_PER = 128
SQ = SQ_PER * N_DEV
D = 512
HQ = 4
DH = 64
HD = HQ * DH
R_HOPS = 4
L_HOPS = 3

_inv = 1.0 / (10000.0 ** (np.arange(0, DH, 2) / DH))
_pos = np.arange(SQ)[:, None] * _inv[None, :]
_cos = np.repeat(np.cos(_pos), 2, axis=-1)
_sin = np.repeat(np.sin(_pos), 2, axis=-1)
_COS = np.tile(_cos, (1, HQ)).astype(np.float32)
_SIN = np.tile(_sin, (1, HQ)).astype(np.float32)
_P = np.zeros((HD, HD), np.float32)
for _blk in range(HQ):
    for _i in range(0, DH, 2):
        _o = _blk * DH
        _P[_o + _i + 1, _o + _i] = -1.0
        _P[_o + _i, _o + _i + 1] = 1.0
_P = _P.astype(np.float32)


def kernel(x, Wq, Wk, Wv, Wo):
    def body(x_ref, wq_ref, wk_ref, wv_ref, wo_ref, cos_ref, sin_ref, p_ref,
             out_ref,
             xch, qf, kf, vf, ctx, rsstage, rsin,
             agRs, agRr, agLs, agLr, rs_s, rs_r):
        my = lax.axis_index("i")
        left = (my + N_DEV - 1) % N_DEV
        right = (my + 1) % N_DEV
        pending = []

        barrier_sem = pltpu.get_barrier_semaphore()
        for nbr in (left, right):
            pl.semaphore_signal(
                barrier_sem, inc=1,
                device_id=(nbr,), device_id_type=pl.DeviceIdType.MESH,
            )
        pl.semaphore_wait(barrier_sem, 2)

        bf16 = jnp.bfloat16
        f32 = jnp.float32
        wq_b = (wq_ref[...] * 0.125).astype(bf16)
        wk_b = wk_ref[...].astype(bf16)
        wv_b = wv_ref[...].astype(bf16)
        wo_b = wo_ref[...].astype(bf16)
        p_b = p_ref[...].astype(bf16)

        def send_chunk(slot_c, sem_arr_s, sem_arr_r, step, dst):
            d = pltpu.make_async_remote_copy(
                src_ref=xch.at[slot_c], dst_ref=xch.at[slot_c],
                send_sem=sem_arr_s.at[step], recv_sem=sem_arr_r.at[step],
                device_id=(dst,), device_id_type=pl.DeviceIdType.MESH,
            )
            d.start()
            pending.append(d)

        def wait_chunk(slot_c, sem_arr_s, sem_arr_r, step, dst):
            d = pltpu.make_async_remote_copy(
                src_ref=xch.at[slot_c], dst_ref=xch.at[slot_c],
                send_sem=sem_arr_s.at[step], recv_sem=sem_arr_r.at[step],
                device_id=(dst,), device_id_type=pl.DeviceIdType.MESH,
            )
            d.wait_recv()

        def qkv_chunk(c):
            rows = pl.ds(c * SQ_PER, SQ_PER)
            cosr = cos_ref[rows, :]
            sinr = sin_ref[rows, :]
            for b in range(B):
                xb = xch[c, b]
                q = jnp.dot(xb, wq_b, preferred_element_type=f32)
                k = jnp.dot(xb, wk_b, preferred_element_type=f32)
                v = jnp.dot(xb, wv_b, preferred_element_type=f32)
                qp = jnp.dot(q.astype(bf16), p_b, preferred_element_type=f32)
                kp = jnp.dot(k.astype(bf16), p_b, preferred_element_type=f32)
                qr = (q * cosr + qp * sinr).astype(bf16)
                kr = (k * cosr + kp * sinr).astype(bf16)
                vb = v.astype(bf16)
                ones_col = (lax.broadcasted_iota(jnp.int32, (SQ_PER, DH), 1)
                            == 0).astype(bf16)
                for h in range(HQ):
                    qf[b, h, rows, :] = qr[:, h * DH:(h + 1) * DH]
                    kf[b, h, rows, :] = kr[:, h * DH:(h + 1) * DH]
                    vf[b, h, rows, 0:DH] = vb[:, h * DH:(h + 1) * DH]
                    vf[b, h, rows, DH:] = ones_col

        xch[my] = x_ref[...].astype(bf16)
        send_chunk(my, agRs, agRr, 0, right)
        send_chunk(my, agLs, agLr, 0, left)
        qkv_chunk(my)
        for s in range(R_HOPS):
            rm = (my + N_DEV - 1 - s) % N_DEV
            wait_chunk(rm, agRs, agRr, s, right)
            if s + 1 < R_HOPS:
                send_chunk(rm, agRs, agRr, s + 1, right)
            lm = (my + 1 + s) % N_DEV
            if s < L_HOPS:
                wait_chunk(lm, agLs, agLr, s, left)
                if s + 1 < L_HOPS:
                    send_chunk(lm, agLs, agLr, s + 1, left)
            qkv_chunk(rm)
            if s < L_HOPS:
                qkv_chunk(lm)

        def att_chunk(c):
            rows = pl.ds(c * SQ_PER, SQ_PER)
            for b in range(B):
                for h in range(HQ):
                    qb = qf[b, h, rows, :]
                    s_ = lax.dot_general(
                        qb, kf[b, h], (((1,), (1,)), ((), ())),
                        preferred_element_type=f32,
                    )
                    e = jnp.exp(s_).astype(bf16)
                    aug = jnp.dot(e, vf[b, h], preferred_element_type=f32)
                    ctx[b, h, rows, :] = (
                        aug[:, :DH] * (1.0 / aug[:, DH:DH + 1])
                    ).astype(bf16)

        def pout(c):
            rows = pl.ds(c * SQ_PER, SQ_PER)
            res = []
            for b in range(B):
                acc = None
                for h in range(HQ):
                    t = jnp.dot(ctx[b, h, rows, :],
                                wo_b[h * DH:(h + 1) * DH, :],
                                preferred_element_type=f32)
                    acc = t if acc is None else acc + t
                res.append(acc)
            return res

        for j in range(1, N_DEV):
            c = (my + j) % N_DEV
            att_chunk(c)
            pc = pout(c)
            rsstage[j - 1, 0] = pc[0].astype(bf16)
            rsstage[j - 1, 1] = pc[1].astype(bf16)
            d = pltpu.make_async_remote_copy(
                src_ref=rsstage.at[j - 1], dst_ref=rsin.at[N_DEV - 1 - j],
                send_sem=rs_s.at[j - 1], recv_sem=rs_r.at[N_DEV - 1 - j],
                device_id=(c,), device_id_type=pl.DeviceIdType.MESH,
            )
            d.start()
            pending.append(d)

        att_chunk(my)
        pm = pout(my)
        acc0, acc1 = pm
        for s in reversed(range(N_DEV - 1)):
            w = pltpu.make_async_remote_copy(
                src_ref=rsstage.at[0], dst_ref=rsin.at[s],
                send_sem=rs_s.at[0], recv_sem=rs_r.at[s],
                device_id=(right,), device_id_type=pl.DeviceIdType.MESH,
            )
            w.wait_recv()
            acc0 = acc0 + rsin[s, 0].astype(f32)
            acc1 = acc1 + rsin[s, 1].astype(f32)
        out_ref[0] = acc0
        out_ref[1] = acc1

        for d in pending:
            d.wait_send()

    cos = jnp.asarray(_COS)
    sin = jnp.asarray(_SIN)
    pmat = jnp.asarray(_P)

    return pl.pallas_call(
        body,
        out_shape=jax.ShapeDtypeStruct((B, SQ_PER, D), jnp.float32),
        in_specs=[pl.BlockSpec(memory_space=pltpu.VMEM)] * 8,
        out_specs=pl.BlockSpec(memory_space=pltpu.VMEM),
        scratch_shapes=[
            pltpu.VMEM((N_DEV, B, SQ_PER, D), jnp.bfloat16),
            pltpu.VMEM((B, HQ, SQ, DH), jnp.bfloat16),
            pltpu.VMEM((B, HQ, SQ, DH), jnp.bfloat16),
            pltpu.VMEM((B, HQ, SQ, 2 * DH), jnp.bfloat16),
            pltpu.VMEM((B, HQ, SQ, DH), jnp.bfloat16),
            pltpu.VMEM((N_DEV - 1, B, SQ_PER, D), jnp.bfloat16),
            pltpu.VMEM((N_DEV - 1, B, SQ_PER, D), jnp.bfloat16),
            pltpu.SemaphoreType.DMA((R_HOPS,)),
            pltpu.SemaphoreType.DMA((R_HOPS,)),
            pltpu.SemaphoreType.DMA((L_HOPS,)),
            pltpu.SemaphoreType.DMA((L_HOPS,)),
            pltpu.SemaphoreType.DMA((N_DEV - 1,)),
            pltpu.SemaphoreType.DMA((N_DEV - 1,)),
        ],
        compiler_params=pltpu.CompilerParams(
            collective_id=0,
            vmem_limit_bytes=100 * 1024 * 1024,
        ),
    )(x, Wq, Wk, Wv, Wo, cos, sin, pmat)
